# baseline (speedup 1.0000x reference)
import functools

import jax
import jax.numpy as jnp
from jax import lax
from jax.experimental import pallas as pl
from jax.experimental.pallas import tpu as pltpu
from jax.experimental.pallas import tpu_sc as plsc

N = 10000
E = 320000
D = 128
DT = 64
NG = 16

NC = 2
NS = 16
NW = NC * NS
CH = 128


def _vmesh():
    return plsc.VectorSubcoreMesh(core_axis_name="c", subcore_axis_name="s")


def _emb_gather(emb, x):
    n = x.shape[0]
    d = emb.shape[1]
    nfull = n // CH
    tail = n - nfull * CH
    tail_w = nfull % NW
    kmax = (nfull + NW - 1) // NW

    @functools.partial(
        pl.kernel,
        mesh=_vmesh(),
        out_type=jax.ShapeDtypeStruct((n, d), jnp.float32),
        scratch_types=(
            [pltpu.VMEM((CH,), jnp.int32) for _ in range(kmax)]
            + [pltpu.VMEM((CH, d), jnp.float32) for _ in range(kmax)]
            + [pltpu.VMEM((max(tail, 8),), jnp.int32),
               pltpu.VMEM((max(tail, 8), d), jnp.float32)]
            + [pltpu.SemaphoreType.DMA for _ in range(3 * kmax + 1)]
        ),
    )
    def k(emb_h, x_h, out_h, *refs):
        idx = refs[0:kmax]
        rows = refs[kmax:2 * kmax]
        tidx_v, trows_v = refs[2 * kmax:2 * kmax + 2]
        sem_i = refs[2 * kmax + 2:3 * kmax + 2]
        sem_g = refs[3 * kmax + 2:4 * kmax + 2]
        sem_o = refs[4 * kmax + 2:5 * kmax + 2]
        sem_t = refs[5 * kmax + 2]

        wid = lax.axis_index("s") * NC + lax.axis_index("c")

        for kk in range(kmax):
            @pl.when(wid + kk * NW < nfull)
            def _(kk=kk):
                off = (wid + kk * NW) * CH
                pltpu.async_copy(x_h.at[pl.ds(off, CH)], idx[kk], sem_i[kk])

        for kk in range(kmax):
            @pl.when(wid + kk * NW < nfull)
            def _(kk=kk):
                off = (wid + kk * NW) * CH
                pltpu.make_async_copy(x_h.at[pl.ds(off, CH)], idx[kk],
                                      sem_i[kk]).wait()
                pltpu.async_copy(emb_h.at[idx[kk]], rows[kk], sem_g[kk])

        for kk in range(kmax):
            @pl.when(wid + kk * NW < nfull)
            def _(kk=kk):
                off = (wid + kk * NW) * CH
                pltpu.make_async_copy(emb_h.at[idx[kk]], rows[kk],
                                      sem_g[kk]).wait()
                pltpu.async_copy(rows[kk], out_h.at[pl.ds(off, CH)],
                                 sem_o[kk])

        if tail:
            @pl.when(wid == tail_w)
            def _():
                off = nfull * CH
                pltpu.sync_copy(x_h.at[pl.ds(off, tail)],
                                tidx_v.at[pl.ds(0, tail)])
                pltpu.async_copy(emb_h.at[tidx_v.at[pl.ds(0, tail)]],
                                 trows_v.at[pl.ds(0, tail)], sem_t).wait()
                pltpu.sync_copy(trows_v.at[pl.ds(0, tail)],
                                out_h.at[pl.ds(off, tail)])

        for kk in range(kmax):
            @pl.when(wid + kk * NW < nfull)
            def _(kk=kk):
                off = (wid + kk * NW) * CH
                pltpu.make_async_copy(rows[kk], out_h.at[pl.ds(off, CH)],
                                      sem_o[kk]).wait()

    return k(emb, x)


def _edge_agg(h, eidx):
    CHE = 104
    per_tile = E // NW
    nfull = per_tile // CHE
    tail = per_tile - nfull * CHE
    nzc = N // CHE
    rz = N - nzc * CHE
    rz_w = nzc % NS

    NB = 3
    NR = 3

    @functools.partial(
        pl.kernel,
        mesh=_vmesh(),
        out_type=jax.ShapeDtypeStruct((NC, N, D), jnp.float32),
        scratch_types=(
            [pltpu.VMEM((CHE,), jnp.int32) for _ in range(NB)]
            + [pltpu.VMEM((1, CHE), jnp.int32) for _ in range(NB)]
            + [pltpu.VMEM((CHE, D), jnp.float32) for _ in range(NR)]
            + [
                pltpu.VMEM((max(tail, 8),), jnp.int32),
                pltpu.VMEM((1, max(tail, 8)), jnp.int32),
                pltpu.VMEM((max(tail, 8), D), jnp.float32),
                pltpu.VMEM_SHARED((N, D), jnp.float32),
            ]
            + [pltpu.SemaphoreType.DMA for _ in range(NB + 2 * NR + 1)]
        ),
    )
    def k(h_h, eidx_h, out_h, *refs):
        sidx = refs[0:NB]
        didx = refs[NB:2 * NB]
        rows = refs[2 * NB:2 * NB + NR]
        tsidx, tdidx, trows, agg_sh = refs[2 * NB + NR:2 * NB + NR + 4]
        off0 = 2 * NB + NR + 4
        sem_i = refs[off0:off0 + NB]
        sem_g = refs[off0 + NB:off0 + NB + NR]
        sem_s = refs[off0 + NB + NR:off0 + NB + 2 * NR]
        sem_t = refs[off0 + NB + 2 * NR]

        cid = lax.axis_index("c")
        sid = lax.axis_index("s")
        wid = sid * NC + cid

        ebase = wid * per_tile

        def idx_start(c, b):
            off = ebase + c * CHE
            pltpu.async_copy(eidx_h.at[pl.ds(off, CHE)], sidx[b], sem_i[b])
            pltpu.async_copy(eidx_h.at[pl.ds(E + off, CHE)], didx[b].at[0],
                             sem_i[b])

        def idx_wait(c, b):
            off = ebase + c * CHE
            pltpu.make_async_copy(eidx_h.at[pl.ds(off, CHE)], sidx[b],
                                  sem_i[b]).wait()
            pltpu.make_async_copy(eidx_h.at[pl.ds(E + off, CHE)],
                                  didx[b].at[0], sem_i[b]).wait()

        def gather_start(b, r):
            pltpu.async_copy(h_h.at[sidx[b]], rows[r], sem_g[r])

        def gather_wait(b, r):
            pltpu.make_async_copy(h_h.at[sidx[b]], rows[r], sem_g[r]).wait()

        def scat_start(b, r):
            pltpu.async_copy(rows[r], agg_sh.at[didx[b].at[0]], sem_s[r],
                             add=True)

        def scat_wait(b, r):
            pltpu.make_async_copy(rows[r], agg_sh.at[didx[b].at[0]],
                                  sem_s[r]).wait()

        def step(c_dyn, c_stat):
            b, r = c_stat % NB, c_stat % NR
            b1, r1 = (c_stat + 1) % NB, (c_stat + 1) % NR
            if c_stat + 2 < nfull:
                idx_start(c_dyn + 2, (c_stat + 2) % NB)
            if c_stat >= 2:
                scat_wait((c_stat - 2) % NB, (c_stat - 2) % NR)
            if c_stat + 1 < nfull:
                idx_wait(c_dyn + 1, b1)
                gather_start(b1, r1)
            gather_wait(b, r)
            scat_start(b, r)

        idx_start(0, 0)
        idx_start(1, 1)

        @pl.loop(0, CHE)
        def _(i):
            @pl.loop(0, D, step=16)
            def _(j):
                rows[2][i, pl.ds(j, 16)] = jnp.zeros((16,), jnp.float32)

        idx_wait(0, 0)
        gather_start(0, 0)

        nzpt = nzc // NS
        for zk in range(nzpt):
            pltpu.async_copy(
                rows[2], agg_sh.at[pl.ds((sid + zk * NS) * CHE, CHE)], sem_t)
        if rz:
            @pl.when(sid == rz_w)
            def _():
                pltpu.sync_copy(rows[2].at[pl.ds(0, rz)],
                                agg_sh.at[pl.ds(nzc * CHE, rz)])
        for zk in range(nzpt):
            pltpu.make_async_copy(
                rows[2], agg_sh.at[pl.ds((sid + zk * NS) * CHE, CHE)],
                sem_t).wait()
        plsc.subcore_barrier()

        for c0 in range(2):
            step(c0, c0)

        @pl.loop(2, nfull - 4, step=NB)
        def _(g):
            for k_ in range(NB):
                step(g + k_, 2 + k_)

        for c0 in range(nfull - 4, nfull):
            step(c0, c0)

        if tail:
            off = ebase + nfull * CHE
            pltpu.sync_copy(eidx_h.at[pl.ds(off, tail)],
                            tsidx.at[pl.ds(0, tail)])
            pltpu.sync_copy(eidx_h.at[pl.ds(E + off, tail)],
                            tdidx.at[0, pl.ds(0, tail)])
            pltpu.async_copy(h_h.at[tsidx.at[pl.ds(0, tail)]],
                             trows.at[pl.ds(0, tail)], sem_t).wait()
            pltpu.sync_copy(trows.at[pl.ds(0, tail)],
                            agg_sh.at[tdidx.at[0, pl.ds(0, tail)]], add=True)

        scat_wait((nfull - 2) % NB, (nfull - 2) % NR)
        scat_wait((nfull - 1) % NB, (nfull - 1) % NR)

        plsc.subcore_barrier()

        nzpt2 = nzc // NS
        for zk in range(nzpt2):
            r0 = (sid + zk * NS) * CHE
            pltpu.async_copy(agg_sh.at[pl.ds(r0, CHE)],
                             out_h.at[cid, pl.ds(r0, CHE)], sem_t)
        if rz:
            @pl.when(sid == rz_w)
            def _():
                pltpu.sync_copy(agg_sh.at[pl.ds(nzc * CHE, rz)],
                                out_h.at[cid, pl.ds(nzc * CHE, rz)])
        for zk in range(nzpt2):
            r0 = (sid + zk * NS) * CHE
            pltpu.make_async_copy(agg_sh.at[pl.ds(r0, CHE)],
                                  out_h.at[cid, pl.ds(r0, CHE)],
                                  sem_t).wait()

    return k(h, eidx)


def _dot_t(a, w):
    return lax.dot_general(a, w, (((1,), (1,)), ((), ())),
                           preferred_element_type=jnp.float32)


def _mlp(z, w1, b1, g1, be1, w2, b2, g2, be2):
    ones_row = jnp.ones((1, z.shape[0]), jnp.float32)

    def bn_relu(t, g, b):
        s1 = jnp.dot(ones_row, t, preferred_element_type=jnp.float32)
        s2 = jnp.dot(ones_row, t * t, preferred_element_type=jnp.float32)
        m = s1 * (1.0 / N)
        v = s2 * (1.0 / N) - m * m
        return jnp.maximum((t - m) * lax.rsqrt(v + 1e-5) * g + b, 0.0)

    h = bn_relu(_dot_t(z, w1) + b1, g1, be1)
    return bn_relu(_dot_t(h, w2) + b2, g2, be2)


def _pool(h, batch_row):
    onehot = (batch_row == lax.broadcasted_iota(jnp.int32, (NG, N), 0))
    onehot = onehot.astype(jnp.float32)
    cnt = jnp.sum(onehot, axis=1, keepdims=True)
    s = jnp.dot(onehot, h, preferred_element_type=jnp.float32)
    return s / jnp.maximum(cnt, 1.0), cnt


def _stage1(h0, w1t, b1, g1, be1, w2t, b2, g2, be2, l0wt, l0b, batch2d):
    def body(h0_r, w1t_r, b1_r, g1_r, be1_r, w2t_r, b2_r, g2_r, be2_r,
             l0wt_r, l0b_r, bat_r, h1_o, out0_o):
        h1 = _mlp(h0_r[...], w1t_r[...], b1_r[...], g1_r[...], be1_r[...],
                  w2t_r[...], b2_r[...], g2_r[...], be2_r[...])
        h1_o[...] = h1
        pm, cnt = _pool(h1, bat_r[...])
        mask = (cnt > 0.0).astype(jnp.float32)
        out0_o[...] = _dot_t(pm, l0wt_r[...]) + mask * l0b_r[...]

    return pl.pallas_call(
        body,
        out_shape=(jax.ShapeDtypeStruct((N, D), jnp.float32),
                   jax.ShapeDtypeStruct((NG, DT), jnp.float32)),
    )(h0, w1t, b1, g1, be1, w2t, b2, g2, be2, l0wt, l0b, batch2d)


def _stage_mid(h_prev, parts, w1t, b1, g1, be1, w2t, b2, g2, be2,
               lwt, lb, batch2d, acc):
    def body(h_r, p_r, w1t_r, b1_r, g1_r, be1_r, w2t_r, b2_r, g2_r, be2_r,
             lwt_r, lb_r, bat_r, acc_r, h_o, acc_o):
        hin = h_r[...] + p_r[0] + p_r[1]
        h = _mlp(hin, w1t_r[...], b1_r[...], g1_r[...], be1_r[...],
                 w2t_r[...], b2_r[...], g2_r[...], be2_r[...])
        h_o[...] = h
        pm, _ = _pool(h, bat_r[...])
        acc_o[...] = acc_r[...] + _dot_t(pm, lwt_r[...]) + lb_r[...]

    return pl.pallas_call(
        body,
        out_shape=(jax.ShapeDtypeStruct((N, D), jnp.float32),
                   jax.ShapeDtypeStruct((NG, DT), jnp.float32)),
    )(h_prev, parts, w1t, b1, g1, be1, w2t, b2, g2, be2, lwt, lb, batch2d,
      acc)


def _stage_last(h_prev, parts, w1t, b1, g1, be1, w2t, b2, g2, be2,
                lwt, lb, batch2d, acc):
    def body(h_r, p_r, w1t_r, b1_r, g1_r, be1_r, w2t_r, b2_r, g2_r, be2_r,
             lwt_r, lb_r, bat_r, acc_r, out_o):
        hin = h_r[...] + p_r[0] + p_r[1]
        h = _mlp(hin, w1t_r[...], b1_r[...], g1_r[...], be1_r[...],
                 w2t_r[...], b2_r[...], g2_r[...], be2_r[...])
        pm, _ = _pool(h, bat_r[...])
        o = acc_r[...] + _dot_t(pm, lwt_r[...]) + lb_r[...]
        m = jnp.max(o, axis=-1, keepdims=True)
        e = jnp.exp(o - m)
        out_o[...] = e / jnp.sum(e, axis=-1, keepdims=True)

    return pl.pallas_call(
        body,
        out_shape=jax.ShapeDtypeStruct((NG, DT), jnp.float32),
    )(h_prev, parts, w1t, b1, g1, be1, w2t, b2, g2, be2, lwt, lb, batch2d,
      acc)


def kernel(x, edge_index, batch, emb,
           fh_W1, fh_b1, fh_g1, fh_be1, fh_W2, fh_b2, fh_g2, fh_be2,
           c1_W1, c1_b1, c1_g1, c1_be1, c1_W2, c1_b2, c1_g2, c1_be2,
           c2_W1, c2_b1, c2_g1, c2_be1, c2_W2, c2_b2, c2_g2, c2_be2,
           lin0_W, lin0_b, lin1_W, lin1_b, lin2_W, lin2_b):
    eidx = edge_index.reshape(2 * E)
    batch2d = batch.reshape(1, N)
    r = lambda v: v.reshape(1, -1)

    h0 = _emb_gather(emb, x)
    h1, out0 = _stage1(h0, fh_W1, r(fh_b1), r(fh_g1), r(fh_be1),
                       fh_W2, r(fh_b2), r(fh_g2), r(fh_be2),
                       lin0_W, r(lin0_b), batch2d)
    p1 = _edge_agg(h1, eidx)
    h2, acc = _stage_mid(h1, p1, c1_W1, r(c1_b1), r(c1_g1), r(c1_be1),
                         c1_W2, r(c1_b2), r(c1_g2), r(c1_be2),
                         lin1_W, r(lin1_b), batch2d, out0)
    p2 = _edge_agg(h2, eidx)
    return _stage_last(h2, p2, c2_W1, r(c2_b1), r(c2_g1), r(c2_be1),
                       c2_W2, r(c2_b2), r(c2_g2), r(c2_be2),
                       lin2_W, r(lin2_b), batch2d, acc)


# speedup vs baseline: 1.0068x; 1.0068x over previous
import functools

import jax
import jax.numpy as jnp
from jax import lax
from jax.experimental import pallas as pl
from jax.experimental.pallas import tpu as pltpu
from jax.experimental.pallas import tpu_sc as plsc

N = 10000
E = 320000
D = 128
DT = 64
NG = 16

NC = 2
NS = 16
NW = NC * NS
CH = 128


def _vmesh():
    return plsc.VectorSubcoreMesh(core_axis_name="c", subcore_axis_name="s")


def _emb_gather(emb, x):
    n = x.shape[0]
    d = emb.shape[1]
    nfull = n // CH
    tail = n - nfull * CH
    tail_w = nfull % NW
    kmax = (nfull + NW - 1) // NW

    @functools.partial(
        pl.kernel,
        mesh=_vmesh(),
        out_type=jax.ShapeDtypeStruct((n, d), jnp.float32),
        scratch_types=(
            [pltpu.VMEM((CH,), jnp.int32) for _ in range(kmax)]
            + [pltpu.VMEM((CH, d), jnp.float32) for _ in range(kmax)]
            + [pltpu.VMEM((max(tail, 8),), jnp.int32),
               pltpu.VMEM((max(tail, 8), d), jnp.float32)]
            + [pltpu.SemaphoreType.DMA for _ in range(3 * kmax + 1)]
        ),
    )
    def k(emb_h, x_h, out_h, *refs):
        idx = refs[0:kmax]
        rows = refs[kmax:2 * kmax]
        tidx_v, trows_v = refs[2 * kmax:2 * kmax + 2]
        sem_i = refs[2 * kmax + 2:3 * kmax + 2]
        sem_g = refs[3 * kmax + 2:4 * kmax + 2]
        sem_o = refs[4 * kmax + 2:5 * kmax + 2]
        sem_t = refs[5 * kmax + 2]

        wid = lax.axis_index("s") * NC + lax.axis_index("c")

        for kk in range(kmax):
            @pl.when(wid + kk * NW < nfull)
            def _(kk=kk):
                off = (wid + kk * NW) * CH
                pltpu.async_copy(x_h.at[pl.ds(off, CH)], idx[kk], sem_i[kk])

        for kk in range(kmax):
            @pl.when(wid + kk * NW < nfull)
            def _(kk=kk):
                off = (wid + kk * NW) * CH
                pltpu.make_async_copy(x_h.at[pl.ds(off, CH)], idx[kk],
                                      sem_i[kk]).wait()
                pltpu.async_copy(emb_h.at[idx[kk]], rows[kk], sem_g[kk])

        for kk in range(kmax):
            @pl.when(wid + kk * NW < nfull)
            def _(kk=kk):
                off = (wid + kk * NW) * CH
                pltpu.make_async_copy(emb_h.at[idx[kk]], rows[kk],
                                      sem_g[kk]).wait()
                pltpu.async_copy(rows[kk], out_h.at[pl.ds(off, CH)],
                                 sem_o[kk])

        if tail:
            @pl.when(wid == tail_w)
            def _():
                off = nfull * CH
                pltpu.sync_copy(x_h.at[pl.ds(off, tail)],
                                tidx_v.at[pl.ds(0, tail)])
                pltpu.async_copy(emb_h.at[tidx_v.at[pl.ds(0, tail)]],
                                 trows_v.at[pl.ds(0, tail)], sem_t).wait()
                pltpu.sync_copy(trows_v.at[pl.ds(0, tail)],
                                out_h.at[pl.ds(off, tail)])

        for kk in range(kmax):
            @pl.when(wid + kk * NW < nfull)
            def _(kk=kk):
                off = (wid + kk * NW) * CH
                pltpu.make_async_copy(rows[kk], out_h.at[pl.ds(off, CH)],
                                      sem_o[kk]).wait()

    return k(emb, x)


def _edge_agg(h, eidx):
    CHE = 104
    per_tile = E // NW
    nfull = per_tile // CHE
    tail = per_tile - nfull * CHE
    nzc = N // CHE
    rz = N - nzc * CHE
    rz_w = nzc % NS

    NB = 3
    NR = 3

    @functools.partial(
        pl.kernel,
        mesh=_vmesh(),
        out_type=jax.ShapeDtypeStruct((NC, N, D), jnp.float32),
        scratch_types=(
            [pltpu.VMEM((CHE,), jnp.int32) for _ in range(NB)]
            + [pltpu.VMEM((1, CHE), jnp.int32) for _ in range(NB)]
            + [pltpu.VMEM((CHE, D), jnp.float32) for _ in range(NR)]
            + [
                pltpu.VMEM((max(tail, 8),), jnp.int32),
                pltpu.VMEM((1, max(tail, 8)), jnp.int32),
                pltpu.VMEM((max(tail, 8), D), jnp.float32),
                pltpu.VMEM_SHARED((N, D), jnp.float32),
            ]
            + [pltpu.SemaphoreType.DMA for _ in range(NB + 2 * NR + 1)]
        ),
    )
    def k(h_h, eidx_h, out_h, *refs):
        sidx = refs[0:NB]
        didx = refs[NB:2 * NB]
        rows = refs[2 * NB:2 * NB + NR]
        tsidx, tdidx, trows, agg_sh = refs[2 * NB + NR:2 * NB + NR + 4]
        off0 = 2 * NB + NR + 4
        sem_i = refs[off0:off0 + NB]
        sem_g = refs[off0 + NB:off0 + NB + NR]
        sem_s = refs[off0 + NB + NR:off0 + NB + 2 * NR]
        sem_t = refs[off0 + NB + 2 * NR]

        cid = lax.axis_index("c")
        sid = lax.axis_index("s")
        wid = sid * NC + cid

        ebase = wid * per_tile

        def idx_start(c, b):
            off = ebase + c * CHE
            pltpu.async_copy(eidx_h.at[pl.ds(off, CHE)], sidx[b], sem_i[b])
            pltpu.async_copy(eidx_h.at[pl.ds(E + off, CHE)], didx[b].at[0],
                             sem_i[b])

        def idx_wait(c, b):
            off = ebase + c * CHE
            pltpu.make_async_copy(eidx_h.at[pl.ds(off, CHE)], sidx[b],
                                  sem_i[b]).wait()
            pltpu.make_async_copy(eidx_h.at[pl.ds(E + off, CHE)],
                                  didx[b].at[0], sem_i[b]).wait()

        def gather_start(b, r):
            pltpu.async_copy(h_h.at[sidx[b]], rows[r], sem_g[r])

        def gather_wait(b, r):
            pltpu.make_async_copy(h_h.at[sidx[b]], rows[r], sem_g[r]).wait()

        def scat_start(b, r):
            pltpu.async_copy(rows[r], agg_sh.at[didx[b].at[0]], sem_s[r],
                             add=True)

        def scat_wait(b, r):
            pltpu.make_async_copy(rows[r], agg_sh.at[didx[b].at[0]],
                                  sem_s[r]).wait()

        def step(c_dyn, c_stat):
            b, r = c_stat % NB, c_stat % NR
            b1, r1 = (c_stat + 1) % NB, (c_stat + 1) % NR
            if c_stat + 2 < nfull:
                idx_start(c_dyn + 2, (c_stat + 2) % NB)
            if c_stat >= 2:
                scat_wait((c_stat - 2) % NB, (c_stat - 2) % NR)
            if c_stat + 1 < nfull:
                idx_wait(c_dyn + 1, b1)
                gather_start(b1, r1)
            gather_wait(b, r)
            scat_start(b, r)

        idx_start(0, 0)
        idx_start(1, 1)

        @pl.loop(0, CHE)
        def _(i):
            @pl.loop(0, D, step=16)
            def _(j):
                rows[2][i, pl.ds(j, 16)] = jnp.zeros((16,), jnp.float32)

        idx_wait(0, 0)
        gather_start(0, 0)

        nzpt = nzc // NS
        for zk in range(nzpt):
            pltpu.async_copy(
                rows[2], agg_sh.at[pl.ds((sid + zk * NS) * CHE, CHE)], sem_t)
        if rz:
            @pl.when(sid == rz_w)
            def _():
                pltpu.sync_copy(rows[2].at[pl.ds(0, rz)],
                                agg_sh.at[pl.ds(nzc * CHE, rz)])
        for zk in range(nzpt):
            pltpu.make_async_copy(
                rows[2], agg_sh.at[pl.ds((sid + zk * NS) * CHE, CHE)],
                sem_t).wait()
        plsc.subcore_barrier()

        for c0 in range(2):
            step(c0, c0)

        @pl.loop(2, nfull - 4, step=NB)
        def _(g):
            for k_ in range(NB):
                step(g + k_, 2 + k_)

        for c0 in range(nfull - 4, nfull):
            step(c0, c0)

        if tail:
            off = ebase + nfull * CHE
            pltpu.sync_copy(eidx_h.at[pl.ds(off, tail)],
                            tsidx.at[pl.ds(0, tail)])
            pltpu.sync_copy(eidx_h.at[pl.ds(E + off, tail)],
                            tdidx.at[0, pl.ds(0, tail)])
            pltpu.async_copy(h_h.at[tsidx.at[pl.ds(0, tail)]],
                             trows.at[pl.ds(0, tail)], sem_t).wait()
            pltpu.sync_copy(trows.at[pl.ds(0, tail)],
                            agg_sh.at[tdidx.at[0, pl.ds(0, tail)]], add=True)

        scat_wait((nfull - 2) % NB, (nfull - 2) % NR)
        scat_wait((nfull - 1) % NB, (nfull - 1) % NR)

        plsc.subcore_barrier()

        @pl.loop(sid, nzc, step=NS)
        def _(i):
            pltpu.sync_copy(agg_sh.at[pl.ds(i * CHE, CHE)],
                            out_h.at[cid, pl.ds(i * CHE, CHE)])

        if rz:
            @pl.when(sid == rz_w)
            def _():
                pltpu.sync_copy(agg_sh.at[pl.ds(nzc * CHE, rz)],
                                out_h.at[cid, pl.ds(nzc * CHE, rz)])

    return k(h, eidx)


def _dot_t(a, w):
    return lax.dot_general(a, w, (((1,), (1,)), ((), ())),
                           preferred_element_type=jnp.float32)


def _mlp(z, w1, b1, g1, be1, w2, b2, g2, be2):
    ones_row = jnp.ones((1, z.shape[0]), jnp.float32)

    def bn_relu(t, g, b):
        s1 = jnp.dot(ones_row, t, preferred_element_type=jnp.float32)
        s2 = jnp.dot(ones_row, t * t, preferred_element_type=jnp.float32)
        m = s1 * (1.0 / N)
        v = s2 * (1.0 / N) - m * m
        return jnp.maximum((t - m) * lax.rsqrt(v + 1e-5) * g + b, 0.0)

    h = bn_relu(_dot_t(z, w1) + b1, g1, be1)
    return bn_relu(_dot_t(h, w2) + b2, g2, be2)


def _pool(h, batch_row):
    onehot = (batch_row == lax.broadcasted_iota(jnp.int32, (NG, N), 0))
    onehot = onehot.astype(jnp.float32)
    cnt = jnp.sum(onehot, axis=1, keepdims=True)
    s = jnp.dot(onehot, h, preferred_element_type=jnp.float32)
    return s / jnp.maximum(cnt, 1.0), cnt


def _stage1(h0, w1t, b1, g1, be1, w2t, b2, g2, be2, l0wt, l0b, batch2d):
    def body(h0_r, w1t_r, b1_r, g1_r, be1_r, w2t_r, b2_r, g2_r, be2_r,
             l0wt_r, l0b_r, bat_r, h1_o, out0_o):
        h1 = _mlp(h0_r[...], w1t_r[...], b1_r[...], g1_r[...], be1_r[...],
                  w2t_r[...], b2_r[...], g2_r[...], be2_r[...])
        h1_o[...] = h1
        pm, cnt = _pool(h1, bat_r[...])
        mask = (cnt > 0.0).astype(jnp.float32)
        out0_o[...] = _dot_t(pm, l0wt_r[...]) + mask * l0b_r[...]

    return pl.pallas_call(
        body,
        out_shape=(jax.ShapeDtypeStruct((N, D), jnp.float32),
                   jax.ShapeDtypeStruct((NG, DT), jnp.float32)),
    )(h0, w1t, b1, g1, be1, w2t, b2, g2, be2, l0wt, l0b, batch2d)


def _stage_mid(h_prev, parts, w1t, b1, g1, be1, w2t, b2, g2, be2,
               lwt, lb, batch2d, acc):
    def body(h_r, p_r, w1t_r, b1_r, g1_r, be1_r, w2t_r, b2_r, g2_r, be2_r,
             lwt_r, lb_r, bat_r, acc_r, h_o, acc_o):
        hin = h_r[...] + p_r[0] + p_r[1]
        h = _mlp(hin, w1t_r[...], b1_r[...], g1_r[...], be1_r[...],
                 w2t_r[...], b2_r[...], g2_r[...], be2_r[...])
        h_o[...] = h
        pm, _ = _pool(h, bat_r[...])
        acc_o[...] = acc_r[...] + _dot_t(pm, lwt_r[...]) + lb_r[...]

    return pl.pallas_call(
        body,
        out_shape=(jax.ShapeDtypeStruct((N, D), jnp.float32),
                   jax.ShapeDtypeStruct((NG, DT), jnp.float32)),
    )(h_prev, parts, w1t, b1, g1, be1, w2t, b2, g2, be2, lwt, lb, batch2d,
      acc)


def _stage_last(h_prev, parts, w1t, b1, g1, be1, w2t, b2, g2, be2,
                lwt, lb, batch2d, acc):
    def body(h_r, p_r, w1t_r, b1_r, g1_r, be1_r, w2t_r, b2_r, g2_r, be2_r,
             lwt_r, lb_r, bat_r, acc_r, out_o):
        hin = h_r[...] + p_r[0] + p_r[1]
        h = _mlp(hin, w1t_r[...], b1_r[...], g1_r[...], be1_r[...],
                 w2t_r[...], b2_r[...], g2_r[...], be2_r[...])
        pm, _ = _pool(h, bat_r[...])
        o = acc_r[...] + _dot_t(pm, lwt_r[...]) + lb_r[...]
        m = jnp.max(o, axis=-1, keepdims=True)
        e = jnp.exp(o - m)
        out_o[...] = e / jnp.sum(e, axis=-1, keepdims=True)

    return pl.pallas_call(
        body,
        out_shape=jax.ShapeDtypeStruct((NG, DT), jnp.float32),
    )(h_prev, parts, w1t, b1, g1, be1, w2t, b2, g2, be2, lwt, lb, batch2d,
      acc)


def kernel(x, edge_index, batch, emb,
           fh_W1, fh_b1, fh_g1, fh_be1, fh_W2, fh_b2, fh_g2, fh_be2,
           c1_W1, c1_b1, c1_g1, c1_be1, c1_W2, c1_b2, c1_g2, c1_be2,
           c2_W1, c2_b1, c2_g1, c2_be1, c2_W2, c2_b2, c2_g2, c2_be2,
           lin0_W, lin0_b, lin1_W, lin1_b, lin2_W, lin2_b):
    eidx = edge_index.reshape(2 * E)
    batch2d = batch.reshape(1, N)
    r = lambda v: v.reshape(1, -1)

    h0 = _emb_gather(emb, x)
    h1, out0 = _stage1(h0, fh_W1, r(fh_b1), r(fh_g1), r(fh_be1),
                       fh_W2, r(fh_b2), r(fh_g2), r(fh_be2),
                       lin0_W, r(lin0_b), batch2d)
    p1 = _edge_agg(h1, eidx)
    h2, acc = _stage_mid(h1, p1, c1_W1, r(c1_b1), r(c1_g1), r(c1_be1),
                         c1_W2, r(c1_b2), r(c1_g2), r(c1_be2),
                         lin1_W, r(lin1_b), batch2d, out0)
    p2 = _edge_agg(h2, eidx)
    return _stage_last(h2, p2, c2_W1, r(c2_b1), r(c2_g1), r(c2_be1),
                       c2_W2, r(c2_b2), r(c2_g2), r(c2_be2),
                       lin2_W, r(lin2_b), batch2d, acc)
